# Initial kernel scaffold; baseline (speedup 1.0000x reference)
import jax
import jax.numpy as jnp
from jax.experimental import pallas as pl


def kernel(x, edge_index, W1l, b1, W1r, W2l, b2, W2r, Wlin, blin):
    raise NotImplementedError("write your pallas kernel here")


# speedup vs baseline: 4.9221x; 4.9221x over previous
import functools

import jax
import jax.numpy as jnp
from jax import lax
from jax.experimental import pallas as pl
from jax.experimental.pallas import tpu as pltpu
from jax.experimental.pallas import tpu_sc as plsc

N = 10000
E = 320000
D = 128

NC = 2
NS = 16
NW = NC * NS
EPT = E // NW
CH = 80
NCHUNK = EPT // CH
RCH = 80
NRCH = N // RCH
DEGW = 16


def _make_sc_agg(dw):
    mesh = plsc.VectorSubcoreMesh(core_axis_name="c", subcore_axis_name="s")
    out_type = [jax.ShapeDtypeStruct((NC, N, dw), jnp.float32)]
    scratch = [
        pltpu.VMEM((NCHUNK, CH), jnp.int32),
        pltpu.VMEM((CH,), jnp.int32),
        pltpu.VMEM((CH, dw), jnp.float32),
        pltpu.VMEM_SHARED((N, dw), jnp.float32),
        pltpu.SemaphoreType.DMA,
    ]

    @functools.partial(pl.kernel, mesh=mesh, out_type=out_type,
                       scratch_types=scratch)
    def sc_agg(feat, src, dst, out, src_v, dst_ch, rows_v, acc, sem):
        c = lax.axis_index("c")
        s = lax.axis_index("s")
        wid = c * NS + s

        zero16 = jnp.zeros((16,), jnp.float32)
        for i in range(RCH):
            for k in range(dw // 16):
                rows_v[i, pl.ds(k * 16, 16)] = zero16

        def zchunk(j, _):
            k = s + j * NS

            @pl.when(k < NRCH)
            def _():
                pltpu.sync_copy(rows_v, acc.at[pl.ds(k * RCH, RCH)])
            return 0
        lax.fori_loop(0, (NRCH + NS - 1) // NS, zchunk, 0)

        plsc.subcore_barrier()

        pltpu.sync_copy(src.at[wid], src_v)

        def step(j, _):
            pltpu.sync_copy(dst.at[wid, j], dst_ch)
            pltpu.async_copy(feat.at[src_v.at[j]], rows_v, sem).wait()
            pltpu.sync_copy(rows_v, acc.at[dst_ch], add=True)
            return 0
        lax.fori_loop(0, NCHUNK, step, 0)

        plsc.subcore_barrier()

        def ochunk(j, _):
            k = s + j * NS

            @pl.when(k < NRCH)
            def _():
                pltpu.sync_copy(acc.at[pl.ds(k * RCH, RCH)],
                                out.at[c, pl.ds(k * RCH, RCH)])
            return 0
        lax.fori_loop(0, (NRCH + NS - 1) // NS, ochunk, 0)

    return sc_agg


def _make_sc_deg():
    mesh = plsc.VectorSubcoreMesh(core_axis_name="c", subcore_axis_name="s")
    out_type = [jax.ShapeDtypeStruct((NC, N, D), jnp.float32)]
    scratch = [
        pltpu.VMEM((CH,), jnp.int32),
        pltpu.VMEM((CH, D), jnp.float32),
        pltpu.VMEM_SHARED((N, D), jnp.float32),
    ]

    @functools.partial(pl.kernel, mesh=mesh, out_type=out_type,
                       scratch_types=scratch)
    def sc_deg(dst, out, dst_ch, ones_v, acc):
        c = lax.axis_index("c")
        s = lax.axis_index("s")
        wid = c * NS + s

        zero16 = jnp.zeros((16,), jnp.float32)
        one16 = jnp.full((16,), 1.0, jnp.float32)
        for i in range(RCH):
            for k in range(D // 16):
                ones_v[i, pl.ds(k * 16, 16)] = zero16

        def zchunk(j, _):
            k = s + j * NS

            @pl.when(k < NRCH)
            def _():
                pltpu.sync_copy(ones_v, acc.at[pl.ds(k * RCH, RCH)])
            return 0
        lax.fori_loop(0, (NRCH + NS - 1) // NS, zchunk, 0)

        for i in range(RCH):
            for k in range(D // 16):
                ones_v[i, pl.ds(k * 16, 16)] = one16

        plsc.subcore_barrier()

        def step(j, _):
            pltpu.sync_copy(dst.at[wid, j], dst_ch)
            pltpu.sync_copy(ones_v, acc.at[dst_ch], add=True)
            return 0
        lax.fori_loop(0, NCHUNK, step, 0)

        plsc.subcore_barrier()

        def ochunk(j, _):
            k = s + j * NS

            @pl.when(k < NRCH)
            def _():
                pltpu.sync_copy(acc.at[pl.ds(k * RCH, RCH)],
                                out.at[c, pl.ds(k * RCH, RCH)])
            return 0
        lax.fori_loop(0, (NRCH + NS - 1) // NS, ochunk, 0)

    return sc_deg


_sc_agg = _make_sc_agg(D)
_sc_deg = _make_sc_deg()


def _first(x):
    return x[0] if isinstance(x, (tuple, list)) else x


BN = 1000


def _tc1_body(p_ref, pd_ref, x_ref, wl_ref, wr_ref, b_ref, h_ref, dinv_ref):
    psum = p_ref[0] + p_ref[1]
    degl = pd_ref[0] + pd_ref[1]
    deg = jnp.sum(degl, axis=1) * (1.0 / D)
    inv = 1.0 / jnp.maximum(deg, 1.0)
    agg = psum * inv[:, None]
    h = (jnp.dot(agg, wl_ref[...], preferred_element_type=jnp.float32)
         + jnp.dot(x_ref[...], wr_ref[...], preferred_element_type=jnp.float32)
         + b_ref[...])
    h_ref[...] = jnp.maximum(h, 0.0)
    dinv_ref[...] = jnp.broadcast_to(inv[:, None], (BN, DEGW))


def _tc1(p, pd, x, wl_t, wr_t, b):
    return pl.pallas_call(
        _tc1_body,
        grid=(N // BN,),
        in_specs=[
            pl.BlockSpec((NC, BN, D), lambda i: (0, i, 0)),
            pl.BlockSpec((NC, BN, D), lambda i: (0, i, 0)),
            pl.BlockSpec((BN, D), lambda i: (i, 0)),
            pl.BlockSpec((D, D), lambda i: (0, 0)),
            pl.BlockSpec((D, D), lambda i: (0, 0)),
            pl.BlockSpec((1, D), lambda i: (0, 0)),
        ],
        out_specs=[
            pl.BlockSpec((BN, D), lambda i: (i, 0)),
            pl.BlockSpec((BN, DEGW), lambda i: (i, 0)),
        ],
        out_shape=[
            jax.ShapeDtypeStruct((N, D), jnp.float32),
            jax.ShapeDtypeStruct((N, DEGW), jnp.float32),
        ],
    )(p, pd, x, wl_t, wr_t, b)


def _tc2_body(p_ref, h_ref, dinv_ref, wl_ref, wr_ref, b_ref, wlin_ref,
              blin_ref, o_ref):
    psum = p_ref[0] + p_ref[1]
    inv = jnp.max(dinv_ref[...], axis=1)
    agg = psum * inv[:, None]
    z = (jnp.dot(agg, wl_ref[...], preferred_element_type=jnp.float32)
         + jnp.dot(h_ref[...], wr_ref[...], preferred_element_type=jnp.float32)
         + b_ref[...])
    z = jnp.maximum(z, 0.0)
    o_ref[...] = (jnp.dot(z, wlin_ref[...], preferred_element_type=jnp.float32)
                  + blin_ref[...])


def _tc2(p, h, dinv, wl_t, wr_t, b, wlin_t, blin):
    return pl.pallas_call(
        _tc2_body,
        grid=(N // BN,),
        in_specs=[
            pl.BlockSpec((NC, BN, D), lambda i: (0, i, 0)),
            pl.BlockSpec((BN, D), lambda i: (i, 0)),
            pl.BlockSpec((BN, DEGW), lambda i: (i, 0)),
            pl.BlockSpec((D, D), lambda i: (0, 0)),
            pl.BlockSpec((D, D), lambda i: (0, 0)),
            pl.BlockSpec((1, D), lambda i: (0, 0)),
            pl.BlockSpec((D, 1), lambda i: (0, 0)),
            pl.BlockSpec((1, 1), lambda i: (0, 0)),
        ],
        out_specs=pl.BlockSpec((BN, 1), lambda i: (i, 0)),
        out_shape=jax.ShapeDtypeStruct((N, 1), jnp.float32),
    )(p, h, dinv, wl_t, wr_t, b, wlin_t, blin)


def kernel(x, edge_index, W1l, b1, W1r, W2l, b2, W2r, Wlin, blin):
    src = edge_index[0].reshape(NW, NCHUNK, CH)
    dst = edge_index[1].reshape(NW, NCHUNK, CH)
    pd = _first(_sc_deg(dst))
    x_dep, _ = lax.optimization_barrier((x, pd))
    p1 = _first(_sc_agg(x_dep, src, dst))
    h, dinv = _tc1(p1, pd, x, W1l.T, W1r.T, b1.reshape(1, D))
    p2 = _first(_sc_agg(h, src, dst))
    out = _tc2(p2, h, dinv, W2l.T, W2r.T, b2.reshape(1, D),
               Wlin.T, blin.reshape(1, 1))
    return out


# speedup vs baseline: 7.2943x; 1.4819x over previous
import functools

import jax
import jax.numpy as jnp
from jax import lax
from jax.experimental import pallas as pl
from jax.experimental.pallas import tpu as pltpu
from jax.experimental.pallas import tpu_sc as plsc

N = 10000
E = 320000
D = 128

NC = 2
NS = 16
NW = NC * NS
EPT = E // NW
CH = 80
NCHUNK = EPT // CH
RCH = 80
NRCH = N // RCH
DEGW = 16


def _make_sc_agg(dw):
    mesh = plsc.VectorSubcoreMesh(core_axis_name="c", subcore_axis_name="s")
    out_type = [jax.ShapeDtypeStruct((NC, N, dw), jnp.float32)]
    scratch = [
        pltpu.VMEM((NCHUNK, CH), jnp.int32),
        pltpu.VMEM((CH,), jnp.int32),
        pltpu.VMEM((CH,), jnp.int32),
        pltpu.VMEM((CH, dw), jnp.float32),
        pltpu.VMEM((CH, dw), jnp.float32),
        pltpu.VMEM_SHARED((N, dw), jnp.float32),
        pltpu.SemaphoreType.DMA,
        pltpu.SemaphoreType.DMA,
    ]

    @functools.partial(pl.kernel, mesh=mesh, out_type=out_type,
                       scratch_types=scratch)
    def sc_agg(feat, src, dst, out, src_v, dc0, dc1, rows0, rows1, acc,
               sem0, sem1):
        c = lax.axis_index("c")
        s = lax.axis_index("s")
        wid = c * NS + s

        zero16 = jnp.zeros((16,), jnp.float32)
        for i in range(RCH):
            for k in range(dw // 16):
                rows0[i, pl.ds(k * 16, 16)] = zero16

        def zchunk(j, _):
            k = s + j * NS

            @pl.when(k < NRCH)
            def _():
                pltpu.sync_copy(rows0, acc.at[pl.ds(k * RCH, RCH)])
            return 0
        lax.fori_loop(0, (NRCH + NS - 1) // NS, zchunk, 0)

        plsc.subcore_barrier()

        pltpu.sync_copy(src.at[wid], src_v)

        slots = ((dc0, rows0, sem0), (dc1, rows1, sem1))
        pltpu.sync_copy(dst.at[wid, 0], dc0)
        pltpu.async_copy(feat.at[src_v.at[0]], rows0, sem0)
        pltpu.sync_copy(dst.at[wid, 1], dc1)
        pltpu.async_copy(feat.at[src_v.at[1]], rows1, sem1)

        def pair(g, _):
            for b in range(2):
                j = g * 2 + b
                dc, rows, sem = slots[b]
                pltpu.make_async_copy(feat.at[src_v.at[j]], rows, sem).wait()
                pltpu.sync_copy(rows, acc.at[dc], add=True)

                @pl.when(j + 2 < NCHUNK)
                def _():
                    pltpu.sync_copy(dst.at[wid, j + 2], dc)
                    pltpu.async_copy(feat.at[src_v.at[j + 2]], rows, sem)
            return 0
        lax.fori_loop(0, (NCHUNK - 1) // 2, pair, 0)
        if NCHUNK % 2 == 1:
            jlast = NCHUNK - 1
            pltpu.make_async_copy(feat.at[src_v.at[jlast]], rows0, sem0).wait()
            pltpu.sync_copy(rows0, acc.at[dc0], add=True)

        plsc.subcore_barrier()

        def ochunk(j, _):
            k = s + j * NS

            @pl.when(k < NRCH)
            def _():
                pltpu.sync_copy(acc.at[pl.ds(k * RCH, RCH)],
                                out.at[c, pl.ds(k * RCH, RCH)])
            return 0
        lax.fori_loop(0, (NRCH + NS - 1) // NS, ochunk, 0)

    return sc_agg


def _make_sc_deg():
    mesh = plsc.VectorSubcoreMesh(core_axis_name="c", subcore_axis_name="s")
    out_type = [jax.ShapeDtypeStruct((NC, N, D), jnp.float32)]
    scratch = [
        pltpu.VMEM((CH,), jnp.int32),
        pltpu.VMEM((CH, D), jnp.float32),
        pltpu.VMEM_SHARED((N, D), jnp.float32),
    ]

    @functools.partial(pl.kernel, mesh=mesh, out_type=out_type,
                       scratch_types=scratch)
    def sc_deg(dst, out, dst_ch, ones_v, acc):
        c = lax.axis_index("c")
        s = lax.axis_index("s")
        wid = c * NS + s

        zero16 = jnp.zeros((16,), jnp.float32)
        one16 = jnp.full((16,), 1.0, jnp.float32)
        for i in range(RCH):
            for k in range(D // 16):
                ones_v[i, pl.ds(k * 16, 16)] = zero16

        def zchunk(j, _):
            k = s + j * NS

            @pl.when(k < NRCH)
            def _():
                pltpu.sync_copy(ones_v, acc.at[pl.ds(k * RCH, RCH)])
            return 0
        lax.fori_loop(0, (NRCH + NS - 1) // NS, zchunk, 0)

        for i in range(RCH):
            for k in range(D // 16):
                ones_v[i, pl.ds(k * 16, 16)] = one16

        plsc.subcore_barrier()

        def step(j, _):
            pltpu.sync_copy(dst.at[wid, j], dst_ch)
            pltpu.sync_copy(ones_v, acc.at[dst_ch], add=True)
            return 0
        lax.fori_loop(0, NCHUNK, step, 0)

        plsc.subcore_barrier()

        def ochunk(j, _):
            k = s + j * NS

            @pl.when(k < NRCH)
            def _():
                pltpu.sync_copy(acc.at[pl.ds(k * RCH, RCH)],
                                out.at[c, pl.ds(k * RCH, RCH)])
            return 0
        lax.fori_loop(0, (NRCH + NS - 1) // NS, ochunk, 0)

    return sc_deg


_sc_agg = _make_sc_agg(D)
_sc_deg = _make_sc_deg()


def _first(x):
    return x[0] if isinstance(x, (tuple, list)) else x


BN = 1000


def _tc1_body(p_ref, pd_ref, x_ref, wl_ref, wr_ref, b_ref, h_ref, dinv_ref):
    psum = p_ref[0] + p_ref[1]
    degl = pd_ref[0] + pd_ref[1]
    deg = jnp.sum(degl, axis=1) * (1.0 / D)
    inv = 1.0 / jnp.maximum(deg, 1.0)
    agg = psum * inv[:, None]
    h = (jnp.dot(agg, wl_ref[...], preferred_element_type=jnp.float32)
         + jnp.dot(x_ref[...], wr_ref[...], preferred_element_type=jnp.float32)
         + b_ref[...])
    h_ref[...] = jnp.maximum(h, 0.0)
    dinv_ref[...] = jnp.broadcast_to(inv[:, None], (BN, DEGW))


def _tc1(p, pd, x, wl_t, wr_t, b):
    return pl.pallas_call(
        _tc1_body,
        grid=(N // BN,),
        in_specs=[
            pl.BlockSpec((NC, BN, D), lambda i: (0, i, 0)),
            pl.BlockSpec((NC, BN, D), lambda i: (0, i, 0)),
            pl.BlockSpec((BN, D), lambda i: (i, 0)),
            pl.BlockSpec((D, D), lambda i: (0, 0)),
            pl.BlockSpec((D, D), lambda i: (0, 0)),
            pl.BlockSpec((1, D), lambda i: (0, 0)),
        ],
        out_specs=[
            pl.BlockSpec((BN, D), lambda i: (i, 0)),
            pl.BlockSpec((BN, DEGW), lambda i: (i, 0)),
        ],
        out_shape=[
            jax.ShapeDtypeStruct((N, D), jnp.float32),
            jax.ShapeDtypeStruct((N, DEGW), jnp.float32),
        ],
    )(p, pd, x, wl_t, wr_t, b)


def _tc2_body(p_ref, h_ref, dinv_ref, wl_ref, wr_ref, b_ref, wlin_ref,
              blin_ref, o_ref):
    psum = p_ref[0] + p_ref[1]
    inv = jnp.max(dinv_ref[...], axis=1)
    agg = psum * inv[:, None]
    z = (jnp.dot(agg, wl_ref[...], preferred_element_type=jnp.float32)
         + jnp.dot(h_ref[...], wr_ref[...], preferred_element_type=jnp.float32)
         + b_ref[...])
    z = jnp.maximum(z, 0.0)
    o_ref[...] = (jnp.dot(z, wlin_ref[...], preferred_element_type=jnp.float32)
                  + blin_ref[...])


def _tc2(p, h, dinv, wl_t, wr_t, b, wlin_t, blin):
    return pl.pallas_call(
        _tc2_body,
        grid=(N // BN,),
        in_specs=[
            pl.BlockSpec((NC, BN, D), lambda i: (0, i, 0)),
            pl.BlockSpec((BN, D), lambda i: (i, 0)),
            pl.BlockSpec((BN, DEGW), lambda i: (i, 0)),
            pl.BlockSpec((D, D), lambda i: (0, 0)),
            pl.BlockSpec((D, D), lambda i: (0, 0)),
            pl.BlockSpec((1, D), lambda i: (0, 0)),
            pl.BlockSpec((D, 1), lambda i: (0, 0)),
            pl.BlockSpec((1, 1), lambda i: (0, 0)),
        ],
        out_specs=pl.BlockSpec((BN, 1), lambda i: (i, 0)),
        out_shape=jax.ShapeDtypeStruct((N, 1), jnp.float32),
    )(p, h, dinv, wl_t, wr_t, b, wlin_t, blin)


def kernel(x, edge_index, W1l, b1, W1r, W2l, b2, W2r, Wlin, blin):
    src = edge_index[0].reshape(NW, NCHUNK, CH)
    dst = edge_index[1].reshape(NW, NCHUNK, CH)
    pd = _first(_sc_deg(dst))
    x_dep, _ = lax.optimization_barrier((x, pd))
    p1 = _first(_sc_agg(x_dep, src, dst))
    h, dinv = _tc1(p1, pd, x, W1l.T, W1r.T, b1.reshape(1, D))
    p2 = _first(_sc_agg(h, src, dst))
    out = _tc2(p2, h, dinv, W2l.T, W2r.T, b2.reshape(1, D),
               Wlin.T, blin.reshape(1, 1))
    return out


# speedup vs baseline: 8.1938x; 1.1233x over previous
import functools

import jax
import jax.numpy as jnp
from jax import lax
from jax.experimental import pallas as pl
from jax.experimental.pallas import tpu as pltpu
from jax.experimental.pallas import tpu_sc as plsc

N = 10000
E = 320000
D = 128

NC = 2
NS = 16
NW = NC * NS
EPT = E // NW
CH = 80
NCHUNK = EPT // CH
RCH = 80
NRCH = N // RCH
DEGW = 16


def _make_sc_agg(dw):
    mesh = plsc.VectorSubcoreMesh(core_axis_name="c", subcore_axis_name="s")
    out_type = [jax.ShapeDtypeStruct((NC, N, dw), jnp.float32)]
    scratch = [
        pltpu.VMEM((NCHUNK, CH), jnp.int32),
        pltpu.VMEM((CH,), jnp.int32),
        pltpu.VMEM((CH,), jnp.int32),
        pltpu.VMEM((CH, dw), jnp.float32),
        pltpu.VMEM((CH, dw), jnp.float32),
        pltpu.VMEM_SHARED((N, dw), jnp.float32),
        pltpu.SemaphoreType.DMA,
        pltpu.SemaphoreType.DMA,
    ]

    @functools.partial(pl.kernel, mesh=mesh, out_type=out_type,
                       scratch_types=scratch)
    def sc_agg(feat, src, dst, out, src_v, dc0, dc1, rows0, rows1, acc,
               sem0, sem1):
        c = lax.axis_index("c")
        s = lax.axis_index("s")
        wid = c * NS + s

        zero16 = jnp.zeros((16,), jnp.float32)
        for i in range(RCH):
            for k in range(dw // 16):
                rows0[i, pl.ds(k * 16, 16)] = zero16

        def zchunk(j, _):
            k = s + j * NS

            @pl.when(k < NRCH)
            def _():
                pltpu.sync_copy(rows0, acc.at[pl.ds(k * RCH, RCH)])
            return 0
        lax.fori_loop(0, (NRCH + NS - 1) // NS, zchunk, 0)

        plsc.subcore_barrier()

        pltpu.sync_copy(src.at[wid], src_v)

        slots = ((dc0, rows0, sem0), (dc1, rows1, sem1))
        pltpu.sync_copy(dst.at[wid, 0], dc0)
        pltpu.async_copy(feat.at[src_v.at[0]], rows0, sem0)
        pltpu.sync_copy(dst.at[wid, 1], dc1)
        pltpu.async_copy(feat.at[src_v.at[1]], rows1, sem1)

        def pair(g, _):
            for b in range(2):
                j = g * 2 + b
                dc, rows, sem = slots[b]
                pltpu.make_async_copy(feat.at[src_v.at[j]], rows, sem).wait()
                pltpu.sync_copy(rows, acc.at[dc], add=True)

                @pl.when(j + 2 < NCHUNK)
                def _():
                    pltpu.sync_copy(dst.at[wid, j + 2], dc)
                    pltpu.async_copy(feat.at[src_v.at[j + 2]], rows, sem)
            return 0
        lax.fori_loop(0, (NCHUNK - 1) // 2, pair, 0)
        if NCHUNK % 2 == 1:
            jlast = NCHUNK - 1
            pltpu.make_async_copy(feat.at[src_v.at[jlast]], rows0, sem0).wait()
            pltpu.sync_copy(rows0, acc.at[dc0], add=True)

        plsc.subcore_barrier()

        def ochunk(j, _):
            k = s + j * NS

            @pl.when(k < NRCH)
            def _():
                pltpu.sync_copy(acc.at[pl.ds(k * RCH, RCH)],
                                out.at[c, pl.ds(k * RCH, RCH)])
            return 0
        lax.fori_loop(0, (NRCH + NS - 1) // NS, ochunk, 0)

    return sc_agg


def _make_sc_deg():
    mesh = plsc.VectorSubcoreMesh(core_axis_name="c", subcore_axis_name="s")
    out_type = [jax.ShapeDtypeStruct((NC, N, D), jnp.float32)]
    scratch = [
        pltpu.VMEM((CH,), jnp.int32),
        pltpu.VMEM((CH,), jnp.int32),
        pltpu.VMEM((CH, D), jnp.float32),
        pltpu.VMEM_SHARED((N, D), jnp.float32),
        pltpu.SemaphoreType.DMA,
        pltpu.SemaphoreType.DMA,
    ]

    @functools.partial(pl.kernel, mesh=mesh, out_type=out_type,
                       scratch_types=scratch)
    def sc_deg(dst, out, dc0, dc1, ones_v, acc, sem0, sem1):
        c = lax.axis_index("c")
        s = lax.axis_index("s")
        wid = c * NS + s

        zero16 = jnp.zeros((16,), jnp.float32)
        one16 = jnp.full((16,), 1.0, jnp.float32)
        for i in range(RCH):
            for k in range(D // 16):
                ones_v[i, pl.ds(k * 16, 16)] = zero16

        def zchunk(j, _):
            k = s + j * NS

            @pl.when(k < NRCH)
            def _():
                pltpu.sync_copy(ones_v, acc.at[pl.ds(k * RCH, RCH)])
            return 0
        lax.fori_loop(0, (NRCH + NS - 1) // NS, zchunk, 0)

        for i in range(RCH):
            for k in range(D // 16):
                ones_v[i, pl.ds(k * 16, 16)] = one16

        plsc.subcore_barrier()

        slots = ((dc0, sem0), (dc1, sem1))
        pltpu.sync_copy(dst.at[wid, 0], dc0)
        pltpu.async_copy(ones_v, acc.at[dc0], add=True, sem=sem0)
        pltpu.sync_copy(dst.at[wid, 1], dc1)
        pltpu.async_copy(ones_v, acc.at[dc1], add=True, sem=sem1)

        def pair(g, _):
            for b in range(2):
                j = g * 2 + b
                dc, sem = slots[b]
                pltpu.make_async_copy(ones_v, acc.at[dc], sem).wait()

                @pl.when(j + 2 < NCHUNK)
                def _():
                    pltpu.sync_copy(dst.at[wid, j + 2], dc)
                    pltpu.async_copy(ones_v, acc.at[dc], add=True, sem=sem)
            return 0
        lax.fori_loop(0, (NCHUNK - 1) // 2, pair, 0)
        if NCHUNK % 2 == 1:
            pltpu.make_async_copy(ones_v, acc.at[dc0], sem0).wait()

        plsc.subcore_barrier()

        def ochunk(j, _):
            k = s + j * NS

            @pl.when(k < NRCH)
            def _():
                pltpu.sync_copy(acc.at[pl.ds(k * RCH, RCH)],
                                out.at[c, pl.ds(k * RCH, RCH)])
            return 0
        lax.fori_loop(0, (NRCH + NS - 1) // NS, ochunk, 0)

    return sc_deg


_sc_agg = _make_sc_agg(D)
_sc_deg = _make_sc_deg()


def _first(x):
    return x[0] if isinstance(x, (tuple, list)) else x


BN = 1000


def _tc1_body(p_ref, pd_ref, x_ref, wl_ref, wr_ref, b_ref, h_ref, dinv_ref):
    psum = p_ref[0] + p_ref[1]
    degl = pd_ref[0] + pd_ref[1]
    deg = jnp.sum(degl, axis=1) * (1.0 / D)
    inv = 1.0 / jnp.maximum(deg, 1.0)
    agg = psum * inv[:, None]
    h = (jnp.dot(agg, wl_ref[...], preferred_element_type=jnp.float32)
         + jnp.dot(x_ref[...], wr_ref[...], preferred_element_type=jnp.float32)
         + b_ref[...])
    h_ref[...] = jnp.maximum(h, 0.0)
    dinv_ref[...] = jnp.broadcast_to(inv[:, None], (BN, DEGW))


def _tc1(p, pd, x, wl_t, wr_t, b):
    return pl.pallas_call(
        _tc1_body,
        grid=(N // BN,),
        in_specs=[
            pl.BlockSpec((NC, BN, D), lambda i: (0, i, 0)),
            pl.BlockSpec((NC, BN, D), lambda i: (0, i, 0)),
            pl.BlockSpec((BN, D), lambda i: (i, 0)),
            pl.BlockSpec((D, D), lambda i: (0, 0)),
            pl.BlockSpec((D, D), lambda i: (0, 0)),
            pl.BlockSpec((1, D), lambda i: (0, 0)),
        ],
        out_specs=[
            pl.BlockSpec((BN, D), lambda i: (i, 0)),
            pl.BlockSpec((BN, DEGW), lambda i: (i, 0)),
        ],
        out_shape=[
            jax.ShapeDtypeStruct((N, D), jnp.float32),
            jax.ShapeDtypeStruct((N, DEGW), jnp.float32),
        ],
    )(p, pd, x, wl_t, wr_t, b)


def _tc2_body(p_ref, h_ref, dinv_ref, wl_ref, wr_ref, b_ref, wlin_ref,
              blin_ref, o_ref):
    psum = p_ref[0] + p_ref[1]
    inv = jnp.max(dinv_ref[...], axis=1)
    agg = psum * inv[:, None]
    z = (jnp.dot(agg, wl_ref[...], preferred_element_type=jnp.float32)
         + jnp.dot(h_ref[...], wr_ref[...], preferred_element_type=jnp.float32)
         + b_ref[...])
    z = jnp.maximum(z, 0.0)
    o_ref[...] = (jnp.dot(z, wlin_ref[...], preferred_element_type=jnp.float32)
                  + blin_ref[...])


def _tc2(p, h, dinv, wl_t, wr_t, b, wlin_t, blin):
    return pl.pallas_call(
        _tc2_body,
        grid=(N // BN,),
        in_specs=[
            pl.BlockSpec((NC, BN, D), lambda i: (0, i, 0)),
            pl.BlockSpec((BN, D), lambda i: (i, 0)),
            pl.BlockSpec((BN, DEGW), lambda i: (i, 0)),
            pl.BlockSpec((D, D), lambda i: (0, 0)),
            pl.BlockSpec((D, D), lambda i: (0, 0)),
            pl.BlockSpec((1, D), lambda i: (0, 0)),
            pl.BlockSpec((D, 1), lambda i: (0, 0)),
            pl.BlockSpec((1, 1), lambda i: (0, 0)),
        ],
        out_specs=pl.BlockSpec((BN, 1), lambda i: (i, 0)),
        out_shape=jax.ShapeDtypeStruct((N, 1), jnp.float32),
    )(p, h, dinv, wl_t, wr_t, b, wlin_t, blin)


def kernel(x, edge_index, W1l, b1, W1r, W2l, b2, W2r, Wlin, blin):
    src = edge_index[0].reshape(NW, NCHUNK, CH)
    dst = edge_index[1].reshape(NW, NCHUNK, CH)
    pd = _first(_sc_deg(dst))
    x_dep, _ = lax.optimization_barrier((x, pd))
    p1 = _first(_sc_agg(x_dep, src, dst))
    h, dinv = _tc1(p1, pd, x, W1l.T, W1r.T, b1.reshape(1, D))
    p2 = _first(_sc_agg(h, src, dst))
    out = _tc2(p2, h, dinv, W2l.T, W2r.T, b2.reshape(1, D),
               Wlin.T, blin.reshape(1, 1))
    return out
